# Initial kernel scaffold; baseline (speedup 1.0000x reference)
import jax
import jax.numpy as jnp
from jax.experimental import pallas as pl


def kernel(corr_index, rela_ht, rela_ct, nei_index, W_emb, b_emb, W_ih, W_hh, b_ih, b_hh):
    raise NotImplementedError("write your pallas kernel here")


# speedup vs baseline: 1.1184x; 1.1184x over previous
import jax
import jax.numpy as jnp
from jax.experimental import pallas as pl
from jax.experimental.pallas import tpu as pltpu

P = 512
E = 32
H = 64
N = P * P
BLK = 4096


def _sigmoid(x):
    return 0.5 * jnp.tanh(0.5 * x) + 0.5


def _lstm_block(corr_ref, ht_ref, ct_ref, nei_ref,
                w_emb_ref, b_emb_ref, w_i_ref, w_f_ref, w_g_ref, w_o_ref,
                b_i_ref, b_f_ref, b_g_ref, b_o_ref, ones_ref,
                ht_out_ref, ct_out_ref):
    corr = corr_ref[...]
    ht = ht_ref[...]
    ct = ct_ref[...]
    nei = nei_ref[...]

    dn = (((1,), (0,)), ((), ()))

    emb = jnp.maximum(
        jax.lax.dot_general(corr, w_emb_ref[...], dn,
                            preferred_element_type=jnp.float32)
        + b_emb_ref[...], 0.0)

    embh = emb.astype(jnp.bfloat16)
    hth = ht.astype(jnp.bfloat16)

    def gate(w_ref, b_ref):
        w = w_ref[...]
        return (jax.lax.dot_general(embh, w[:E, :], dn,
                                    preferred_element_type=jnp.float32)
                + jax.lax.dot_general(hth, w[E:, :], dn,
                                      preferred_element_type=jnp.float32)
                + b_ref[...])

    i = _sigmoid(gate(w_i_ref, b_i_ref))
    f = _sigmoid(gate(w_f_ref, b_f_ref))
    g = jnp.tanh(gate(w_g_ref, b_g_ref))
    o = _sigmoid(gate(w_o_ref, b_o_ref))

    c_new = f * ct + i * g
    h_new = o * jnp.tanh(c_new)

    mf = jax.lax.dot_general(nei.astype(jnp.bfloat16), ones_ref[...], dn,
                             preferred_element_type=jnp.float32)
    ht_out_ref[...] = ht + mf * (h_new - ht)
    ct_out_ref[...] = ct + mf * (c_new - ct)


def kernel(corr_index, rela_ht, rela_ct, nei_index, W_emb, b_emb, W_ih, W_hh, b_ih, b_hh):
    corr = corr_index.reshape(N, 2)
    ht = rela_ht.reshape(N, H)
    ct = rela_ct.reshape(N, H)
    neif = (nei_index.reshape(N, 1) > 0).astype(jnp.float32)

    w_emb = W_emb.T
    b_emb_r = b_emb.reshape(1, E)
    w_ih = W_ih.T.astype(jnp.bfloat16)
    w_hh = W_hh.T.astype(jnp.bfloat16)
    bias = (b_ih + b_hh).reshape(1, 4 * H)
    w_gates = [jnp.concatenate([w_ih[:, k * H:(k + 1) * H],
                                w_hh[:, k * H:(k + 1) * H]], axis=0)
               for k in range(4)]
    b_gates = [bias[:, k * H:(k + 1) * H] for k in range(4)]
    ones_h = jnp.ones((1, H), dtype=jnp.bfloat16)

    grid = (N // BLK,)
    row_spec = lambda w: pl.BlockSpec((BLK, w), lambda i: (i, 0))
    full_spec = lambda a, b: pl.BlockSpec((a, b), lambda i: (0, 0))

    ht_out, ct_out = pl.pallas_call(
        _lstm_block,
        grid=grid,
        in_specs=[
            row_spec(2),
            row_spec(H),
            row_spec(H),
            row_spec(1),
            full_spec(2, E),
            full_spec(1, E),
            full_spec(E + H, H),
            full_spec(E + H, H),
            full_spec(E + H, H),
            full_spec(E + H, H),
            full_spec(1, H),
            full_spec(1, H),
            full_spec(1, H),
            full_spec(1, H),
            full_spec(1, H),
        ],
        out_specs=[row_spec(H), row_spec(H)],
        out_shape=[
            jax.ShapeDtypeStruct((N, H), jnp.float32),
            jax.ShapeDtypeStruct((N, H), jnp.float32),
        ],
        compiler_params=pltpu.CompilerParams(
            dimension_semantics=("arbitrary",),
        ),
    )(corr, ht, ct, neif, w_emb, b_emb_r, *w_gates, *b_gates, ones_h)

    return ht_out.reshape(P, P, H), ct_out.reshape(P, P, H)
